# Initial kernel scaffold; baseline (speedup 1.0000x reference)
import jax
import jax.numpy as jnp
from jax.experimental import pallas as pl


def kernel(x, edge_index, edge_weights, W1, b1, W2, b2):
    raise NotImplementedError("write your pallas kernel here")


# speedup vs baseline: 6.9414x; 6.9414x over previous
import functools

import jax
import jax.numpy as jnp
from jax import lax
from jax.experimental import pallas as pl
from jax.experimental.pallas import tpu as pltpu
from jax.experimental.pallas import tpu_sc as plsc


def _mm1_body(x_ref, wt_ref, b_ref, o_ref):
    o_ref[...] = jnp.maximum(
        jnp.dot(x_ref[...], wt_ref[...], preferred_element_type=jnp.float32)
        + b_ref[...], 0.0)


def _mm1(x, w1t, b1row):
    n, din = x.shape
    dh = w1t.shape[1]
    r = 1000
    return pl.pallas_call(
        _mm1_body,
        grid=(n // r,),
        in_specs=[
            pl.BlockSpec((r, din), lambda i: (i, 0)),
            pl.BlockSpec((din, dh), lambda i: (0, 0)),
            pl.BlockSpec((1, dh), lambda i: (0, 0)),
        ],
        out_specs=pl.BlockSpec((r, dh), lambda i: (i, 0)),
        out_shape=jax.ShapeDtypeStruct((n, dh), jnp.float32),
    )(x, w1t, b1row)


def _mm2_body(h_ref, a0_ref, a1_ref, wt_ref, b_ref, o_ref):
    acc = h_ref[...] * 1.5 + a0_ref[0] + a1_ref[0]
    o_ref[...] = (
        jnp.dot(acc, wt_ref[...], preferred_element_type=jnp.float32)
        + b_ref[...])


def _mm2(h, agg, w2t, b2row):
    n, dh = h.shape
    dout = w2t.shape[1]
    r = 1000
    return pl.pallas_call(
        _mm2_body,
        grid=(n // r,),
        in_specs=[
            pl.BlockSpec((r, dh), lambda i: (i, 0)),
            pl.BlockSpec((1, r, dh), lambda i: (0, i, 0)),
            pl.BlockSpec((1, r, dh), lambda i: (1, i, 0)),
            pl.BlockSpec((dh, dout), lambda i: (0, 0)),
            pl.BlockSpec((1, dout), lambda i: (0, 0)),
        ],
        out_specs=pl.BlockSpec((r, dout), lambda i: (i, 0)),
        out_shape=jax.ShapeDtypeStruct((n, dout), jnp.float32),
    )(h, agg, agg, w2t, b2row)


_CH = 128


@functools.cache
def _make_scatter(n, d, e):
    info = plsc.get_sparse_core_info()
    nc, ns = info.num_cores, info.num_subcores
    assert e % (_CH * nc) == 0
    chunks_per_core = e // _CH // nc
    nrow_chunks = n // _CH
    row_rem = n - nrow_chunks * _CH
    rem_tile = nrow_chunks % ns
    nf = d // 16
    mesh = plsc.VectorSubcoreMesh(core_axis_name="c", subcore_axis_name="s")

    @functools.partial(
        pl.kernel, mesh=mesh,
        out_type=jax.ShapeDtypeStruct((nc, n, d), jnp.float32),
        scratch_types=[
            pltpu.VMEM((_CH,), jnp.int32),
            pltpu.VMEM((_CH,), jnp.int32),
            pltpu.VMEM((_CH,), jnp.float32),
            pltpu.VMEM((_CH, d), jnp.float32),
            pltpu.VMEM_SHARED((n, d), jnp.float32),
            pltpu.SemaphoreType.DMA,
        ],
    )
    def scatter_k(h_hbm, src_hbm, dst_hbm, ew_hbm, out_hbm,
                  src_v, dst_v, w_v, rows_v, agg_sh, sem):
        c = lax.axis_index("c")
        s = lax.axis_index("s")
        zero = jnp.zeros((16,), jnp.float32)

        def zrow(i, _):
            for f in range(nf):
                rows_v[i, pl.ds(f * 16, 16)] = zero
            return 0
        lax.fori_loop(0, _CH, zrow, 0)

        n_rmine = (nrow_chunks - s + ns - 1) // ns

        def zcp(i, _):
            pltpu.sync_copy(rows_v, agg_sh.at[pl.ds((s + i * ns) * _CH, _CH)])
            return 0
        lax.fori_loop(0, n_rmine, zcp, 0)
        if row_rem:
            @pl.when(s == rem_tile)
            def _():
                pltpu.sync_copy(rows_v.at[pl.ds(0, row_rem)],
                                agg_sh.at[pl.ds(nrow_chunks * _CH, row_rem)])
        plsc.subcore_barrier()

        n_my = (chunks_per_core - s + ns - 1) // ns

        def body(i, _):
            e0 = (c * chunks_per_core + s + i * ns) * _CH
            pltpu.sync_copy(src_hbm.at[pl.ds(e0, _CH)], src_v)
            pltpu.sync_copy(dst_hbm.at[pl.ds(e0, _CH)], dst_v)
            pltpu.sync_copy(ew_hbm.at[pl.ds(e0, _CH)], w_v)
            pltpu.async_copy(h_hbm.at[src_v], rows_v, sem).wait()

            def sgrp(g, _):
                w16 = w_v[pl.ds(g * 16, 16)]
                for j in range(16):
                    w = w16[j]
                    row = g * 16 + j
                    for f in range(nf):
                        sl = pl.ds(f * 16, 16)
                        rows_v[row, sl] = rows_v[row, sl] * w
                return 0
            lax.fori_loop(0, _CH // 16, sgrp, 0)
            pltpu.sync_copy(rows_v, agg_sh.at[dst_v], add=True)
            return 0
        lax.fori_loop(0, n_my, body, 0)

        plsc.subcore_barrier()

        def wcp(i, _):
            r = (s + i * ns) * _CH
            pltpu.sync_copy(agg_sh.at[pl.ds(r, _CH)], rows_v)
            pltpu.sync_copy(rows_v, out_hbm.at[c, pl.ds(r, _CH)])
            return 0
        lax.fori_loop(0, n_rmine, wcp, 0)
        if row_rem:
            @pl.when(s == rem_tile)
            def _():
                r = nrow_chunks * _CH
                pltpu.sync_copy(agg_sh.at[pl.ds(r, row_rem)],
                                rows_v.at[pl.ds(0, row_rem)])
                pltpu.sync_copy(rows_v.at[pl.ds(0, row_rem)],
                                out_hbm.at[c, pl.ds(r, row_rem)])

    return scatter_k


def kernel(x, edge_index, edge_weights, W1, b1, W2, b2):
    n, _ = x.shape
    e = edge_index.shape[1]
    src = edge_index[0].astype(jnp.int32)
    dst = edge_index[1].astype(jnp.int32)
    ew = edge_weights.astype(jnp.float32)

    h = _mm1(x, W1.T, b1.reshape(1, -1))
    agg = _make_scatter(n, h.shape[1], e)(h, src, dst, ew)
    return _mm2(h, agg, W2.T, b2.reshape(1, -1))
